# baseline (speedup 1.0000x reference)
import functools

import jax
import jax.numpy as jnp
from jax import lax
from jax.experimental import pallas as pl
from jax.experimental.pallas import tpu as pltpu


def _round_up(x, m):
    return ((x + m - 1) // m) * m


def _fused_kernel(src_ref, dst_ref, ef_ref, nf_ref, wn_ref, bn_ref,
                  wu_ref, bu_ref, wv_ref, bv_ref, wer_ref, ber_ref,
                  wed_ref, bias_ref, o_ref, u_tab, v_tab, prod3,
                  *, tm, unroll, tn):
    n_nodes = nf_ref.shape[0]
    p_dim = u_tab.shape[2]

    @pl.when(pl.program_id(0) == 0)
    def _build_tables():
        for nc in range(n_nodes // tn):
            nf_c = nf_ref[pl.ds(nc * tn, tn), :]
            n = jnp.dot(nf_c, wn_ref[...], preferred_element_type=jnp.float32)
            n = jnp.maximum(n + bn_ref[...], 0.0)
            u = jnp.dot(n, wu_ref[...],
                        preferred_element_type=jnp.float32) + bu_ref[...]
            v = jnp.dot(n, wv_ref[...],
                        preferred_element_type=jnp.float32) + bv_ref[...]
            u_tab[pl.ds(nc * tn, tn), :, :] = u.reshape(tn, 1, p_dim)
            v_tab[pl.ds(nc * tn, tn), :, :] = v.reshape(tn, 1, p_dim)

    def chunk(c, carry):
        base8 = c * (unroll // 8)
        for k8 in range(unroll // 8):
            g = base8 + k8
            mi0 = g * 8
            for j in range(8):
                mi = mi0 + j
                s = src_ref[0, 0, mi]
                d = dst_ref[0, 0, mi]
                prod3[g, j] = u_tab[s, 0] * v_tab[d, 0]
        return carry

    lax.fori_loop(0, tm // unroll, chunk, 0)

    wer_bf = wer_ref[...].astype(jnp.bfloat16)
    nh = max(4, tm // 2048)
    rows = tm // nh
    ber_bf = ber_ref[...].astype(jnp.bfloat16)
    wed_bf = wed_ref[...].astype(jnp.bfloat16)
    for h in range(nh):
        ef_bf = ef_ref[pl.ds(h * rows, rows), :].astype(jnp.bfloat16)
        x = jnp.dot(ef_bf, wer_bf, preferred_element_type=jnp.float32)
        xr = jnp.maximum(x.astype(jnp.bfloat16) + ber_bf, jnp.bfloat16(0.0))
        es = jnp.dot(xr, wed_bf, preferred_element_type=jnp.float32)
        pair = jnp.sum(prod3[pl.ds(h * (rows // 8), rows // 8), :, :],
                       axis=-1, keepdims=True).reshape(rows, 1)
        total = pair + es
        t3 = jnp.transpose(total.reshape(rows // 128, 128, 1), (0, 2, 1))
        o_ref[0, pl.ds(h * (rows // 128), rows // 128), :] = (
            t3.reshape(rows // 128, 128) + bias_ref[...])


def kernel(wn, bn, we_red, be_red, wu, bu, wv, bv, we_dot, be_dot,
           node_features, edge_features, src_ids, dst_ids,
           *, block_rows=16384, unroll=64, node_block=512):
    N, K = node_features.shape
    D = wn.shape[1]
    P = wu.shape[1]
    E = src_ids.shape[0]

    tm = min(block_rows, _round_up(E, 8))
    E_pad = _round_up(E, tm)

    src = src_ids.astype(jnp.int32)
    dst = dst_ids.astype(jnp.int32)
    ef = edge_features
    if E_pad != E:
        pad = E_pad - E
        src = jnp.pad(src, (0, pad))
        dst = jnp.pad(dst, (0, pad))
        ef = jnp.pad(ef, ((0, pad), (0, 0)))

    T = E_pad // tm
    src2 = src.reshape(T, 1, tm)
    dst2 = dst.reshape(T, 1, tm)
    bias = be_dot.reshape(1, 1)
    tn = min(node_block, N)

    res = lambda i: (0, 0)

    out = pl.pallas_call(
        functools.partial(_fused_kernel, tm=tm, unroll=unroll, tn=tn),
        out_shape=jax.ShapeDtypeStruct((T, tm // 128, 128), jnp.float32),
        grid=(T,),
        in_specs=[
            pl.BlockSpec((1, 1, tm), lambda i: (i, 0, 0),
                         memory_space=pltpu.SMEM),
            pl.BlockSpec((1, 1, tm), lambda i: (i, 0, 0),
                         memory_space=pltpu.SMEM),
            pl.BlockSpec((tm, K), lambda i: (i, 0)),
            pl.BlockSpec((N, K), res),
            pl.BlockSpec((K, D), res),
            pl.BlockSpec((1, D), res),
            pl.BlockSpec((D, P), res),
            pl.BlockSpec((1, P), res),
            pl.BlockSpec((D, P), res),
            pl.BlockSpec((1, P), res),
            pl.BlockSpec((K, D), res),
            pl.BlockSpec((1, D), res),
            pl.BlockSpec((D, 1), res),
            pl.BlockSpec((1, 1), res),
        ],
        out_specs=pl.BlockSpec((1, tm // 128, 128), lambda i: (i, 0, 0)),
        scratch_shapes=[
            pltpu.VMEM((N, 1, P), jnp.float32),
            pltpu.VMEM((N, 1, P), jnp.float32),
            pltpu.VMEM((tm // 8, 8, P), jnp.float32),
        ],
        compiler_params=pltpu.CompilerParams(
            dimension_semantics=("arbitrary",)),
    )(src2, dst2, ef, node_features, wn, bn, wu, bu, wv, bv,
      we_red, be_red, we_dot, bias)
    out = out.reshape(E_pad, 1)
    return out[:E] if E_pad != E else out


# speedup vs baseline: 1.0218x; 1.0218x over previous
import functools

import jax
import jax.numpy as jnp
from jax import lax
from jax.experimental import pallas as pl
from jax.experimental.pallas import tpu as pltpu


def _round_up(x, m):
    return ((x + m - 1) // m) * m


def _fused_kernel(src_ref, dst_ref, ef_ref, nf_ref, wn_ref, bn_ref,
                  wu_ref, bu_ref, wv_ref, bv_ref, wer_ref, ber_ref,
                  wed_ref, bias_ref, o_ref, u_tab, v_tab, prod3,
                  *, tm, unroll, tn):
    n_nodes = nf_ref.shape[0]
    p_dim = u_tab.shape[2]

    @pl.when(pl.program_id(0) == 0)
    def _build_tables():
        for nc in range(n_nodes // tn):
            nf_c = nf_ref[pl.ds(nc * tn, tn), :]
            n = jnp.dot(nf_c, wn_ref[...], preferred_element_type=jnp.float32)
            n = jnp.maximum(n + bn_ref[...], 0.0)
            u = jnp.dot(n, wu_ref[...],
                        preferred_element_type=jnp.float32) + bu_ref[...]
            v = jnp.dot(n, wv_ref[...],
                        preferred_element_type=jnp.float32) + bv_ref[...]
            u_tab[pl.ds(nc * tn, tn), :, :] = u.reshape(tn, 1, p_dim)
            v_tab[pl.ds(nc * tn, tn), :, :] = v.reshape(tn, 1, p_dim)

    def chunk(c, carry):
        base8 = c * (unroll // 8)
        for k8 in range(unroll // 8):
            g = base8 + k8
            mi0 = g * 8
            for j in range(8):
                mi = mi0 + j
                s = src_ref[0, 0, mi]
                d = dst_ref[0, 0, mi]
                prod3[g, j] = u_tab[s, 0] * v_tab[d, 0]
        return carry

    lax.fori_loop(0, tm // unroll, chunk, 0)

    wer_bf = wer_ref[...].astype(jnp.bfloat16)
    nh = max(4, tm // 2048)
    rows = tm // nh
    ber_bf = ber_ref[...].astype(jnp.bfloat16)
    wed_bf = wed_ref[...].astype(jnp.bfloat16)
    for h in range(nh):
        ef_bf = ef_ref[pl.ds(h * rows, rows), :].astype(jnp.bfloat16)
        x = jnp.dot(ef_bf, wer_bf, preferred_element_type=jnp.float32)
        xr = jnp.maximum(x.astype(jnp.bfloat16) + ber_bf, jnp.bfloat16(0.0))
        es = jnp.dot(xr, wed_bf, preferred_element_type=jnp.float32)
        pair = jnp.sum(prod3[pl.ds(h * (rows // 8), rows // 8), :, :],
                       axis=-1, keepdims=True).reshape(rows, 1)
        total = pair + es
        t3 = jnp.transpose(total.reshape(rows // 128, 128, 1), (0, 2, 1))
        o_ref[0, pl.ds(h * (rows // 128), rows // 128), :] = (
            t3.reshape(rows // 128, 128) + bias_ref[...])


def kernel(wn, bn, we_red, be_red, wu, bu, wv, bv, we_dot, be_dot,
           node_features, edge_features, src_ids, dst_ids,
           *, block_rows=16384, unroll=128, node_block=512):
    N, K = node_features.shape
    D = wn.shape[1]
    P = wu.shape[1]
    E = src_ids.shape[0]

    tm = min(block_rows, _round_up(E, 8))
    E_pad = _round_up(E, tm)

    src = src_ids.astype(jnp.int32)
    dst = dst_ids.astype(jnp.int32)
    ef = edge_features
    if E_pad != E:
        pad = E_pad - E
        src = jnp.pad(src, (0, pad))
        dst = jnp.pad(dst, (0, pad))
        ef = jnp.pad(ef, ((0, pad), (0, 0)))

    T = E_pad // tm
    src2 = src.reshape(T, 1, tm)
    dst2 = dst.reshape(T, 1, tm)
    bias = be_dot.reshape(1, 1)
    tn = min(node_block, N)

    res = lambda i: (0, 0)

    out = pl.pallas_call(
        functools.partial(_fused_kernel, tm=tm, unroll=unroll, tn=tn),
        out_shape=jax.ShapeDtypeStruct((T, tm // 128, 128), jnp.float32),
        grid=(T,),
        in_specs=[
            pl.BlockSpec((1, 1, tm), lambda i: (i, 0, 0),
                         memory_space=pltpu.SMEM),
            pl.BlockSpec((1, 1, tm), lambda i: (i, 0, 0),
                         memory_space=pltpu.SMEM),
            pl.BlockSpec((tm, K), lambda i: (i, 0)),
            pl.BlockSpec((N, K), res),
            pl.BlockSpec((K, D), res),
            pl.BlockSpec((1, D), res),
            pl.BlockSpec((D, P), res),
            pl.BlockSpec((1, P), res),
            pl.BlockSpec((D, P), res),
            pl.BlockSpec((1, P), res),
            pl.BlockSpec((K, D), res),
            pl.BlockSpec((1, D), res),
            pl.BlockSpec((D, 1), res),
            pl.BlockSpec((1, 1), res),
        ],
        out_specs=pl.BlockSpec((1, tm // 128, 128), lambda i: (i, 0, 0)),
        scratch_shapes=[
            pltpu.VMEM((N, 1, P), jnp.float32),
            pltpu.VMEM((N, 1, P), jnp.float32),
            pltpu.VMEM((tm // 8, 8, P), jnp.float32),
        ],
        compiler_params=pltpu.CompilerParams(
            dimension_semantics=("arbitrary",)),
    )(src2, dst2, ef, node_features, wn, bn, wu, bu, wv, bv,
      we_red, be_red, we_dot, bias)
    out = out.reshape(E_pad, 1)
    return out[:E] if E_pad != E else out


# speedup vs baseline: 1.0389x; 1.0166x over previous
import functools

import jax
import jax.numpy as jnp
from jax import lax
from jax.experimental import pallas as pl
from jax.experimental.pallas import tpu as pltpu


def _round_up(x, m):
    return ((x + m - 1) // m) * m


def _fused_kernel(src_ref, dst_ref, ef_ref, nf_ref, wn_ref, bn_ref,
                  wu_ref, bu_ref, wv_ref, bv_ref, wer_ref, ber_ref,
                  wed_ref, bias_ref, o_ref, u_tab, v_tab, prod3,
                  *, tm, unroll, tn):
    n_nodes = nf_ref.shape[0]
    p_dim = u_tab.shape[2]

    @pl.when(pl.program_id(0) == 0)
    def _build_tables():
        for nc in range(n_nodes // tn):
            nf_c = nf_ref[pl.ds(nc * tn, tn), :]
            n = jnp.dot(nf_c, wn_ref[...], preferred_element_type=jnp.float32)
            n = jnp.maximum(n + bn_ref[...], 0.0)
            u = jnp.dot(n, wu_ref[...],
                        preferred_element_type=jnp.float32) + bu_ref[...]
            v = jnp.dot(n, wv_ref[...],
                        preferred_element_type=jnp.float32) + bv_ref[...]
            u_tab[pl.ds(nc * tn, tn), :, :] = u.reshape(tn, 1, p_dim)
            v_tab[pl.ds(nc * tn, tn), :, :] = v.reshape(tn, 1, p_dim)

    def chunk(c, carry):
        base8 = c * (unroll // 8)
        for k8 in range(unroll // 8):
            g = base8 + k8
            mi0 = g * 8
            for j in range(8):
                mi = mi0 + j
                s = src_ref[0, 0, mi]
                d = dst_ref[0, 0, mi]
                prod3[g, j] = u_tab[s, 0] * v_tab[d, 0]
        return carry

    lax.fori_loop(0, tm // unroll, chunk, 0)

    wer_bf = wer_ref[...].astype(jnp.bfloat16)
    nh = max(4, tm // 2048)
    rows = tm // nh
    ber_bf = ber_ref[...].astype(jnp.bfloat16)
    wed_bf = wed_ref[...].astype(jnp.bfloat16)
    for h in range(nh):
        ef_bf = ef_ref[pl.ds(h * rows, rows), :].astype(jnp.bfloat16)
        x = jnp.dot(ef_bf, wer_bf, preferred_element_type=jnp.float32)
        xr = jnp.maximum(x.astype(jnp.bfloat16) + ber_bf, jnp.bfloat16(0.0))
        es = jnp.dot(xr, wed_bf, preferred_element_type=jnp.float32)
        pair = jnp.sum(prod3[pl.ds(h * (rows // 8), rows // 8), :, :],
                       axis=-1, keepdims=True).reshape(rows, 1)
        total = pair + es
        t3 = jnp.transpose(total.reshape(rows // 128, 128, 1), (0, 2, 1))
        o_ref[0, pl.ds(h * (rows // 128), rows // 128), :] = (
            t3.reshape(rows // 128, 128) + bias_ref[...])


def kernel(wn, bn, we_red, be_red, wu, bu, wv, bv, we_dot, be_dot,
           node_features, edge_features, src_ids, dst_ids,
           *, block_rows=16384, unroll=256, node_block=512):
    N, K = node_features.shape
    D = wn.shape[1]
    P = wu.shape[1]
    E = src_ids.shape[0]

    tm = min(block_rows, _round_up(E, 8))
    E_pad = _round_up(E, tm)

    src = src_ids.astype(jnp.int32)
    dst = dst_ids.astype(jnp.int32)
    ef = edge_features
    if E_pad != E:
        pad = E_pad - E
        src = jnp.pad(src, (0, pad))
        dst = jnp.pad(dst, (0, pad))
        ef = jnp.pad(ef, ((0, pad), (0, 0)))

    T = E_pad // tm
    src2 = src.reshape(T, 1, tm)
    dst2 = dst.reshape(T, 1, tm)
    bias = be_dot.reshape(1, 1)
    tn = min(node_block, N)

    res = lambda i: (0, 0)

    out = pl.pallas_call(
        functools.partial(_fused_kernel, tm=tm, unroll=unroll, tn=tn),
        out_shape=jax.ShapeDtypeStruct((T, tm // 128, 128), jnp.float32),
        grid=(T,),
        in_specs=[
            pl.BlockSpec((1, 1, tm), lambda i: (i, 0, 0),
                         memory_space=pltpu.SMEM),
            pl.BlockSpec((1, 1, tm), lambda i: (i, 0, 0),
                         memory_space=pltpu.SMEM),
            pl.BlockSpec((tm, K), lambda i: (i, 0)),
            pl.BlockSpec((N, K), res),
            pl.BlockSpec((K, D), res),
            pl.BlockSpec((1, D), res),
            pl.BlockSpec((D, P), res),
            pl.BlockSpec((1, P), res),
            pl.BlockSpec((D, P), res),
            pl.BlockSpec((1, P), res),
            pl.BlockSpec((K, D), res),
            pl.BlockSpec((1, D), res),
            pl.BlockSpec((D, 1), res),
            pl.BlockSpec((1, 1), res),
        ],
        out_specs=pl.BlockSpec((1, tm // 128, 128), lambda i: (i, 0, 0)),
        scratch_shapes=[
            pltpu.VMEM((N, 1, P), jnp.float32),
            pltpu.VMEM((N, 1, P), jnp.float32),
            pltpu.VMEM((tm // 8, 8, P), jnp.float32),
        ],
        compiler_params=pltpu.CompilerParams(
            dimension_semantics=("arbitrary",)),
    )(src2, dst2, ef, node_features, wn, bn, wu, bu, wv, bv,
      we_red, be_red, we_dot, bias)
    out = out.reshape(E_pad, 1)
    return out[:E] if E_pad != E else out


# speedup vs baseline: 1.0408x; 1.0019x over previous
import functools

import jax
import jax.numpy as jnp
from jax import lax
from jax.experimental import pallas as pl
from jax.experimental.pallas import tpu as pltpu


def _round_up(x, m):
    return ((x + m - 1) // m) * m


def _fused_kernel(src_ref, dst_ref, ef_ref, nf_ref, wn_ref, bn_ref,
                  wu_ref, bu_ref, wv_ref, bv_ref, wer_ref, ber_ref,
                  wed_ref, bias_ref, o_ref, u_tab, v_tab, prod3,
                  *, tm, unroll, tn):
    n_nodes = nf_ref.shape[0]
    p_dim = u_tab.shape[2]

    @pl.when(pl.program_id(0) == 0)
    def _build_tables():
        for nc in range(n_nodes // tn):
            nf_c = nf_ref[pl.ds(nc * tn, tn), :]
            n = jnp.dot(nf_c, wn_ref[...], preferred_element_type=jnp.float32)
            n = jnp.maximum(n + bn_ref[...], 0.0)
            u = jnp.dot(n, wu_ref[...],
                        preferred_element_type=jnp.float32) + bu_ref[...]
            v = jnp.dot(n, wv_ref[...],
                        preferred_element_type=jnp.float32) + bv_ref[...]
            u_tab[pl.ds(nc * tn, tn), :, :] = u.reshape(tn, 1, p_dim)
            v_tab[pl.ds(nc * tn, tn), :, :] = v.reshape(tn, 1, p_dim)

    def chunk(c, carry):
        base8 = c * (unroll // 8)
        for k8 in range(unroll // 8):
            g = base8 + k8
            mi0 = g * 8
            for j in range(8):
                mi = mi0 + j
                s = src_ref[0, 0, mi]
                d = dst_ref[0, 0, mi]
                prod3[g, j] = u_tab[s, 0] * v_tab[d, 0]
        return carry

    lax.fori_loop(0, tm // unroll, chunk, 0)

    wer_bf = wer_ref[...].astype(jnp.bfloat16)
    nh = max(4, tm // 2048)
    rows = tm // nh
    ber_bf = ber_ref[...].astype(jnp.bfloat16)
    wed_bf = wed_ref[...].astype(jnp.bfloat16)
    for h in range(nh):
        ef_bf = ef_ref[pl.ds(h * rows, rows), :].astype(jnp.bfloat16)
        x = jnp.dot(ef_bf, wer_bf, preferred_element_type=jnp.float32)
        xr = jnp.maximum(x.astype(jnp.bfloat16) + ber_bf, jnp.bfloat16(0.0))
        es = jnp.dot(xr, wed_bf, preferred_element_type=jnp.float32)
        pair = jnp.sum(prod3[pl.ds(h * (rows // 8), rows // 8), :, :],
                       axis=-1, keepdims=True).reshape(rows, 1)
        total = pair + es
        t3 = jnp.transpose(total.reshape(rows // 128, 128, 1), (0, 2, 1))
        o_ref[0, pl.ds(h * (rows // 128), rows // 128), :] = (
            t3.reshape(rows // 128, 128) + bias_ref[...])


def kernel(wn, bn, we_red, be_red, wu, bu, wv, bv, we_dot, be_dot,
           node_features, edge_features, src_ids, dst_ids,
           *, block_rows=16384, unroll=512, node_block=512):
    N, K = node_features.shape
    D = wn.shape[1]
    P = wu.shape[1]
    E = src_ids.shape[0]

    tm = min(block_rows, _round_up(E, 8))
    E_pad = _round_up(E, tm)

    src = src_ids.astype(jnp.int32)
    dst = dst_ids.astype(jnp.int32)
    ef = edge_features
    if E_pad != E:
        pad = E_pad - E
        src = jnp.pad(src, (0, pad))
        dst = jnp.pad(dst, (0, pad))
        ef = jnp.pad(ef, ((0, pad), (0, 0)))

    T = E_pad // tm
    src2 = src.reshape(T, 1, tm)
    dst2 = dst.reshape(T, 1, tm)
    bias = be_dot.reshape(1, 1)
    tn = min(node_block, N)

    res = lambda i: (0, 0)

    out = pl.pallas_call(
        functools.partial(_fused_kernel, tm=tm, unroll=unroll, tn=tn),
        out_shape=jax.ShapeDtypeStruct((T, tm // 128, 128), jnp.float32),
        grid=(T,),
        in_specs=[
            pl.BlockSpec((1, 1, tm), lambda i: (i, 0, 0),
                         memory_space=pltpu.SMEM),
            pl.BlockSpec((1, 1, tm), lambda i: (i, 0, 0),
                         memory_space=pltpu.SMEM),
            pl.BlockSpec((tm, K), lambda i: (i, 0)),
            pl.BlockSpec((N, K), res),
            pl.BlockSpec((K, D), res),
            pl.BlockSpec((1, D), res),
            pl.BlockSpec((D, P), res),
            pl.BlockSpec((1, P), res),
            pl.BlockSpec((D, P), res),
            pl.BlockSpec((1, P), res),
            pl.BlockSpec((K, D), res),
            pl.BlockSpec((1, D), res),
            pl.BlockSpec((D, 1), res),
            pl.BlockSpec((1, 1), res),
        ],
        out_specs=pl.BlockSpec((1, tm // 128, 128), lambda i: (i, 0, 0)),
        scratch_shapes=[
            pltpu.VMEM((N, 1, P), jnp.float32),
            pltpu.VMEM((N, 1, P), jnp.float32),
            pltpu.VMEM((tm // 8, 8, P), jnp.float32),
        ],
        compiler_params=pltpu.CompilerParams(
            dimension_semantics=("arbitrary",)),
    )(src2, dst2, ef, node_features, wn, bn, wu, bu, wv, bv,
      we_red, be_red, we_dot, bias)
    out = out.reshape(E_pad, 1)
    return out[:E] if E_pad != E else out


# speedup vs baseline: 1.0433x; 1.0024x over previous
import functools

import jax
import jax.numpy as jnp
from jax import lax
from jax.experimental import pallas as pl
from jax.experimental.pallas import tpu as pltpu


def _round_up(x, m):
    return ((x + m - 1) // m) * m


def _fused_kernel(src_ref, dst_ref, ef_ref, nf_ref, wn_ref, bn_ref,
                  wu_ref, bu_ref, wv_ref, bv_ref, wer_ref, ber_ref,
                  wed_ref, bias_ref, o_ref, u_tab, v_tab, prod3,
                  *, tm, unroll, tn):
    n_nodes = nf_ref.shape[0]
    p_dim = u_tab.shape[2]

    @pl.when(pl.program_id(0) == 0)
    def _build_tables():
        for nc in range(n_nodes // tn):
            nf_c = nf_ref[pl.ds(nc * tn, tn), :]
            n = jnp.dot(nf_c, wn_ref[...], preferred_element_type=jnp.float32)
            n = jnp.maximum(n + bn_ref[...], 0.0)
            u = jnp.dot(n, wu_ref[...],
                        preferred_element_type=jnp.float32) + bu_ref[...]
            v = jnp.dot(n, wv_ref[...],
                        preferred_element_type=jnp.float32) + bv_ref[...]
            u_tab[pl.ds(nc * tn, tn), :, :] = u.reshape(tn, 1, p_dim)
            v_tab[pl.ds(nc * tn, tn), :, :] = v.reshape(tn, 1, p_dim)

    def chunk(c, carry):
        base8 = c * (unroll // 8)
        for k8 in range(unroll // 8):
            g = base8 + k8
            mi0 = g * 8
            for j in range(8):
                mi = mi0 + j
                s = src_ref[0, 0, mi]
                d = dst_ref[0, 0, mi]
                prod3[g, j] = u_tab[s, 0] * v_tab[d, 0]
        return carry

    lax.fori_loop(0, tm // unroll, chunk, 0)

    wer_bf = wer_ref[...].astype(jnp.bfloat16)
    nh = max(4, tm // 2048)
    rows = tm // nh
    ber_bf = ber_ref[...].astype(jnp.bfloat16)
    wed_bf = wed_ref[...].astype(jnp.bfloat16)
    for h in range(nh):
        ef_bf = ef_ref[pl.ds(h * rows, rows), :].astype(jnp.bfloat16)
        x = jnp.dot(ef_bf, wer_bf, preferred_element_type=jnp.float32)
        xr = jnp.maximum(x.astype(jnp.bfloat16) + ber_bf, jnp.bfloat16(0.0))
        es = jnp.dot(xr, wed_bf, preferred_element_type=jnp.float32)
        pair = jnp.sum(prod3[pl.ds(h * (rows // 8), rows // 8), :, :],
                       axis=-1, keepdims=True).reshape(rows, 1)
        total = pair + es
        t3 = jnp.transpose(total.reshape(rows // 128, 128, 1), (0, 2, 1))
        o_ref[0, pl.ds(h * (rows // 128), rows // 128), :] = (
            t3.reshape(rows // 128, 128) + bias_ref[...])


def kernel(wn, bn, we_red, be_red, wu, bu, wv, bv, we_dot, be_dot,
           node_features, edge_features, src_ids, dst_ids,
           *, block_rows=16384, unroll=1024, node_block=512):
    N, K = node_features.shape
    D = wn.shape[1]
    P = wu.shape[1]
    E = src_ids.shape[0]

    tm = min(block_rows, _round_up(E, 8))
    E_pad = _round_up(E, tm)

    src = src_ids.astype(jnp.int32)
    dst = dst_ids.astype(jnp.int32)
    ef = edge_features
    if E_pad != E:
        pad = E_pad - E
        src = jnp.pad(src, (0, pad))
        dst = jnp.pad(dst, (0, pad))
        ef = jnp.pad(ef, ((0, pad), (0, 0)))

    T = E_pad // tm
    src2 = src.reshape(T, 1, tm)
    dst2 = dst.reshape(T, 1, tm)
    bias = be_dot.reshape(1, 1)
    tn = min(node_block, N)

    res = lambda i: (0, 0)

    out = pl.pallas_call(
        functools.partial(_fused_kernel, tm=tm, unroll=unroll, tn=tn),
        out_shape=jax.ShapeDtypeStruct((T, tm // 128, 128), jnp.float32),
        grid=(T,),
        in_specs=[
            pl.BlockSpec((1, 1, tm), lambda i: (i, 0, 0),
                         memory_space=pltpu.SMEM),
            pl.BlockSpec((1, 1, tm), lambda i: (i, 0, 0),
                         memory_space=pltpu.SMEM),
            pl.BlockSpec((tm, K), lambda i: (i, 0)),
            pl.BlockSpec((N, K), res),
            pl.BlockSpec((K, D), res),
            pl.BlockSpec((1, D), res),
            pl.BlockSpec((D, P), res),
            pl.BlockSpec((1, P), res),
            pl.BlockSpec((D, P), res),
            pl.BlockSpec((1, P), res),
            pl.BlockSpec((K, D), res),
            pl.BlockSpec((1, D), res),
            pl.BlockSpec((D, 1), res),
            pl.BlockSpec((1, 1), res),
        ],
        out_specs=pl.BlockSpec((1, tm // 128, 128), lambda i: (i, 0, 0)),
        scratch_shapes=[
            pltpu.VMEM((N, 1, P), jnp.float32),
            pltpu.VMEM((N, 1, P), jnp.float32),
            pltpu.VMEM((tm // 8, 8, P), jnp.float32),
        ],
        compiler_params=pltpu.CompilerParams(
            dimension_semantics=("arbitrary",)),
    )(src2, dst2, ef, node_features, wn, bn, wu, bu, wv, bv,
      we_red, be_red, we_dot, bias)
    out = out.reshape(E_pad, 1)
    return out[:E] if E_pad != E else out
